# baseline (speedup 1.0000x reference)
import jax
import jax.numpy as jnp
from jax.experimental import pallas as pl
from jax.experimental.pallas import tpu as pltpu

_B, _N, _M = 4, 4096, 4096
_TN = 256
_NT = _N // _TN
_C1 = 1000.0 / (2.0 * _B * _N)
_C2 = 1000.0 / (2.0 * _B * _M)


def _chamfer_body(a1_ref, a2t_ref, out_ref, d2_scr):
    b = pl.program_id(0)
    n = pl.program_id(1)

    a1 = a1_ref[0]
    a1x = a1[:, 0:1]
    a1y = a1[:, 1:2]
    a1z = a1[:, 2:3]
    asq = a1x * a1x + a1y * a1y + a1z * a1z

    a2t = a2t_ref[0]
    a2x = a2t[0:1, :]
    a2y = a2t[1:2, :]
    a2z = a2t[2:3, :]
    bsq = a2x * a2x + a2y * a2y + a2z * a2z

    u = jax.lax.dot_general(
        a1.astype(jnp.bfloat16),
        a2t.astype(jnp.bfloat16) * jnp.bfloat16(-2.0),
        (((1,), (0,)), ((), ())),
        preferred_element_type=jnp.float32,
    )
    d = (asq + bsq) + u

    @pl.when(jnp.logical_and(b == 0, n == 0))
    def _():
        out_ref[...] = jnp.zeros((1, 1), jnp.float32)

    d1 = jnp.maximum(jnp.min(d, axis=1, keepdims=True), 0.0)
    s1 = jnp.sum(jnp.sqrt(d1), keepdims=True)

    dmin = jnp.min(d, axis=0, keepdims=True)

    @pl.when(n == 0)
    def _():
        d2_scr[...] = dmin

    @pl.when(n > 0)
    def _():
        d2_scr[...] = jnp.minimum(d2_scr[...], dmin)

    out_ref[...] += s1 * _C1

    @pl.when(n == _NT - 1)
    def _():
        d2 = jnp.maximum(d2_scr[...], 0.0)
        out_ref[...] += jnp.sum(jnp.sqrt(d2), keepdims=True) * _C2


def kernel(array1, array2):
    a2t = jnp.transpose(array2, (0, 2, 1))
    out = pl.pallas_call(
        _chamfer_body,
        grid=(_B, _NT),
        in_specs=[
            pl.BlockSpec((1, _TN, 3), lambda b, n: (b, n, 0)),
            pl.BlockSpec((1, 3, _M), lambda b, n: (b, 0, 0)),
        ],
        out_specs=pl.BlockSpec((1, 1), lambda b, n: (0, 0)),
        out_shape=jax.ShapeDtypeStruct((1, 1), jnp.float32),
        scratch_shapes=[pltpu.VMEM((1, _M), jnp.float32)],
    )(array1, a2t)
    return out[0, 0]


# speedup vs baseline: 1.1409x; 1.1409x over previous
import jax
import jax.numpy as jnp
from jax.experimental import pallas as pl
from jax.experimental.pallas import tpu as pltpu

_B, _N, _M = 4, 4096, 4096
_TN = 512
_NT = _N // _TN
_C1 = 1000.0 / (2.0 * _B * _N)
_C2 = 1000.0 / (2.0 * _B * _M)


def _chamfer_body(a1_ref, a2t_ref, out_ref, d2_scr):
    b = pl.program_id(0)
    n = pl.program_id(1)

    a1 = a1_ref[0]
    a1x = a1[:, 0:1]
    a1y = a1[:, 1:2]
    a1z = a1[:, 2:3]
    asq = a1x * a1x + a1y * a1y + a1z * a1z

    a2t = a2t_ref[0]
    a2x = a2t[0:1, :]
    a2y = a2t[1:2, :]
    a2z = a2t[2:3, :]
    bsq = a2x * a2x + a2y * a2y + a2z * a2z

    u = jax.lax.dot_general(
        a1.astype(jnp.bfloat16),
        a2t.astype(jnp.bfloat16) * jnp.bfloat16(-2.0),
        (((1,), (0,)), ((), ())),
        preferred_element_type=jnp.float32,
    )
    d = (asq + bsq) + u

    @pl.when(jnp.logical_and(b == 0, n == 0))
    def _():
        out_ref[...] = jnp.zeros((1, 1), jnp.float32)

    d1 = jnp.maximum(jnp.min(d, axis=1, keepdims=True), 0.0)
    s1 = jnp.sum(jnp.sqrt(d1), keepdims=True)

    dmin = jnp.min(d, axis=0, keepdims=True)

    @pl.when(n == 0)
    def _():
        d2_scr[...] = dmin

    @pl.when(n > 0)
    def _():
        d2_scr[...] = jnp.minimum(d2_scr[...], dmin)

    out_ref[...] += s1 * _C1

    @pl.when(n == _NT - 1)
    def _():
        d2 = jnp.maximum(d2_scr[...], 0.0)
        out_ref[...] += jnp.sum(jnp.sqrt(d2), keepdims=True) * _C2


def kernel(array1, array2):
    a2t = jnp.transpose(array2, (0, 2, 1))
    out = pl.pallas_call(
        _chamfer_body,
        grid=(_B, _NT),
        in_specs=[
            pl.BlockSpec((1, _TN, 3), lambda b, n: (b, n, 0)),
            pl.BlockSpec((1, 3, _M), lambda b, n: (b, 0, 0)),
        ],
        out_specs=pl.BlockSpec((1, 1), lambda b, n: (0, 0)),
        out_shape=jax.ShapeDtypeStruct((1, 1), jnp.float32),
        scratch_shapes=[pltpu.VMEM((1, _M), jnp.float32)],
    )(array1, a2t)
    return out[0, 0]


# speedup vs baseline: 1.2083x; 1.0590x over previous
import jax
import jax.numpy as jnp
from jax.experimental import pallas as pl
from jax.experimental.pallas import tpu as pltpu

_B, _N, _M = 4, 4096, 4096
_TN = 1024
_NT = _N // _TN
_C1 = 1000.0 / (2.0 * _B * _N)
_C2 = 1000.0 / (2.0 * _B * _M)


def _chamfer_body(a1_ref, a2t_ref, out_ref, d2_scr):
    b = pl.program_id(0)
    n = pl.program_id(1)

    a1 = a1_ref[0]
    a1x = a1[:, 0:1]
    a1y = a1[:, 1:2]
    a1z = a1[:, 2:3]
    asq = a1x * a1x + a1y * a1y + a1z * a1z

    a2t = a2t_ref[0]
    a2x = a2t[0:1, :]
    a2y = a2t[1:2, :]
    a2z = a2t[2:3, :]
    bsq = a2x * a2x + a2y * a2y + a2z * a2z

    u = jax.lax.dot_general(
        a1.astype(jnp.bfloat16),
        a2t.astype(jnp.bfloat16) * jnp.bfloat16(-2.0),
        (((1,), (0,)), ((), ())),
        preferred_element_type=jnp.float32,
    )
    d = (asq + bsq) + u

    @pl.when(jnp.logical_and(b == 0, n == 0))
    def _():
        out_ref[...] = jnp.zeros((1, 1), jnp.float32)

    d1 = jnp.maximum(jnp.min(d, axis=1, keepdims=True), 0.0)
    s1 = jnp.sum(jnp.sqrt(d1), keepdims=True)

    dmin = jnp.min(d, axis=0, keepdims=True)

    @pl.when(n == 0)
    def _():
        d2_scr[...] = dmin

    @pl.when(n > 0)
    def _():
        d2_scr[...] = jnp.minimum(d2_scr[...], dmin)

    out_ref[...] += s1 * _C1

    @pl.when(n == _NT - 1)
    def _():
        d2 = jnp.maximum(d2_scr[...], 0.0)
        out_ref[...] += jnp.sum(jnp.sqrt(d2), keepdims=True) * _C2


def kernel(array1, array2):
    a2t = jnp.transpose(array2, (0, 2, 1))
    out = pl.pallas_call(
        _chamfer_body,
        grid=(_B, _NT),
        in_specs=[
            pl.BlockSpec((1, _TN, 3), lambda b, n: (b, n, 0)),
            pl.BlockSpec((1, 3, _M), lambda b, n: (b, 0, 0)),
        ],
        out_specs=pl.BlockSpec((1, 1), lambda b, n: (0, 0)),
        out_shape=jax.ShapeDtypeStruct((1, 1), jnp.float32),
        scratch_shapes=[pltpu.VMEM((1, _M), jnp.float32)],
    )(array1, a2t)
    return out[0, 0]


# speedup vs baseline: 1.2432x; 1.0289x over previous
import jax
import jax.numpy as jnp
from jax.experimental import pallas as pl
from jax.experimental.pallas import tpu as pltpu

_B, _N, _M = 4, 4096, 4096
_TN = 2048
_NT = _N // _TN
_C1 = 1000.0 / (2.0 * _B * _N)
_C2 = 1000.0 / (2.0 * _B * _M)


def _chamfer_body(a1_ref, a2t_ref, out_ref, d2_scr):
    b = pl.program_id(0)
    n = pl.program_id(1)

    a1 = a1_ref[0]
    a1x = a1[:, 0:1]
    a1y = a1[:, 1:2]
    a1z = a1[:, 2:3]
    asq = a1x * a1x + a1y * a1y + a1z * a1z

    a2t = a2t_ref[0]
    a2x = a2t[0:1, :]
    a2y = a2t[1:2, :]
    a2z = a2t[2:3, :]
    bsq = a2x * a2x + a2y * a2y + a2z * a2z

    u = jax.lax.dot_general(
        a1.astype(jnp.bfloat16),
        a2t.astype(jnp.bfloat16) * jnp.bfloat16(-2.0),
        (((1,), (0,)), ((), ())),
        preferred_element_type=jnp.float32,
    )
    d = (asq + bsq) + u

    @pl.when(jnp.logical_and(b == 0, n == 0))
    def _():
        out_ref[...] = jnp.zeros((1, 1), jnp.float32)

    d1 = jnp.maximum(jnp.min(d, axis=1, keepdims=True), 0.0)
    s1 = jnp.sum(jnp.sqrt(d1), keepdims=True)

    dmin = jnp.min(d, axis=0, keepdims=True)

    @pl.when(n == 0)
    def _():
        d2_scr[...] = dmin

    @pl.when(n > 0)
    def _():
        d2_scr[...] = jnp.minimum(d2_scr[...], dmin)

    out_ref[...] += s1 * _C1

    @pl.when(n == _NT - 1)
    def _():
        d2 = jnp.maximum(d2_scr[...], 0.0)
        out_ref[...] += jnp.sum(jnp.sqrt(d2), keepdims=True) * _C2


def kernel(array1, array2):
    a2t = jnp.transpose(array2, (0, 2, 1))
    out = pl.pallas_call(
        _chamfer_body,
        grid=(_B, _NT),
        in_specs=[
            pl.BlockSpec((1, _TN, 3), lambda b, n: (b, n, 0)),
            pl.BlockSpec((1, 3, _M), lambda b, n: (b, 0, 0)),
        ],
        out_specs=pl.BlockSpec((1, 1), lambda b, n: (0, 0)),
        out_shape=jax.ShapeDtypeStruct((1, 1), jnp.float32),
        scratch_shapes=[pltpu.VMEM((1, _M), jnp.float32)],
    )(array1, a2t)
    return out[0, 0]
